# baseline (speedup 1.0000x reference)
import jax
import jax.numpy as jnp
from jax.experimental import pallas as pl
from jax.experimental.pallas import tpu as pltpu

_TOKENS = 32768
_HIDDEN = 768
_E = 64
_K = 8
_BT = 4096


def _router_body(x_ref, wt_ref, logits_ref, wi_ref):
    x = x_ref[...]
    wt = wt_ref[...]
    logits = jax.lax.dot_general(
        x, wt, (((1,), (0,)), ((), ())),
        preferred_element_type=jnp.float32,
    )
    logits_ref[...] = logits

    lt = jax.lax.dot_general(
        wt, x, (((0,), (1,)), ((), ())),
        preferred_element_type=jnp.float32,
    )

    ikey = lt.view(jnp.int32)
    ikey = jnp.bitwise_xor(ikey, jnp.right_shift(ikey, 31) & jnp.int32(0x7FFFFFFF))
    eidx = jax.lax.broadcasted_iota(jnp.int32, (_E, _BT), 0)

    sel_v, sel_i = [], []
    cur = ikey
    neg_inf = jnp.int32(-0x80000000)
    for _ in range(_K):
        ck, ci = cur, eidx
        r = _E
        while r > 1:
            r //= 2
            ak, bk = ck[:r], ck[r:]
            ai, bi = ci[:r], ci[r:]
            take = ak >= bk
            ck = jnp.where(take, ak, bk)
            ci = jnp.where(take, ai, bi)
        m, ix = ck, ci
        sel_v.append(m)
        sel_i.append(ix)
        cur = jnp.where(eidx == ix, neg_inf, cur)
    vk = jnp.concatenate(sel_v, axis=0)
    idx = jnp.concatenate(sel_i, axis=0)

    vals = jnp.bitwise_xor(
        vk, jnp.right_shift(vk, 31) & jnp.int32(0x7FFFFFFF)
    ).view(jnp.float32)

    e = jnp.exp(vals - vals[0:1])
    w = e / jnp.sum(e, axis=0, keepdims=True)

    wi_ref[...] = jnp.concatenate([w, idx.astype(jnp.float32)], axis=0)


@jax.jit
def kernel(hidden_states, weight):
    wt = weight.T
    grid = (_TOKENS // _BT,)
    logits, wi_t = pl.pallas_call(
        _router_body,
        grid=grid,
        in_specs=[
            pl.BlockSpec((_BT, _HIDDEN), lambda i: (i, 0)),
            pl.BlockSpec((_HIDDEN, _E), lambda i: (0, 0)),
        ],
        out_specs=[
            pl.BlockSpec((_BT, _E), lambda i: (i, 0)),
            pl.BlockSpec((2 * _K, _BT), lambda i: (0, i)),
        ],
        out_shape=[
            jax.ShapeDtypeStruct((_TOKENS, _E), jnp.float32),
            jax.ShapeDtypeStruct((2 * _K, _TOKENS), jnp.float32),
        ],
        compiler_params=pltpu.CompilerParams(
            dimension_semantics=("arbitrary",),
        ),
    )(hidden_states, wt)
    wi = wi_t.T
    return (logits, wi[:, :_K], wi[:, _K:].astype(jnp.int32))


# speedup vs baseline: 1.0598x; 1.0598x over previous
import jax
import jax.numpy as jnp
from jax.experimental import pallas as pl
from jax.experimental.pallas import tpu as pltpu

_TOKENS = 32768
_HIDDEN = 768
_E = 64
_K = 8
_BT = 4096


def _router_body(x_ref, wt_ref, logits_ref, wi_ref):
    x = x_ref[...]
    wt = wt_ref[...]
    logits = jax.lax.dot_general(
        x, wt, (((1,), (0,)), ((), ())),
        preferred_element_type=jnp.float32,
    )
    logits_ref[...] = logits

    lt = logits.T

    ikey = lt.view(jnp.int32)
    ikey = jnp.bitwise_xor(ikey, jnp.right_shift(ikey, 31) & jnp.int32(0x7FFFFFFF))
    eidx = jax.lax.broadcasted_iota(jnp.int32, (_E, _BT), 0)

    sel_v, sel_i = [], []
    cur = ikey
    neg_inf = jnp.int32(-0x80000000)
    for _ in range(_K):
        ck, ci = cur, eidx
        r = _E
        while r > 1:
            r //= 2
            ak, bk = ck[:r], ck[r:]
            ai, bi = ci[:r], ci[r:]
            take = ak >= bk
            ck = jnp.where(take, ak, bk)
            ci = jnp.where(take, ai, bi)
        m, ix = ck, ci
        sel_v.append(m)
        sel_i.append(ix)
        cur = jnp.where(eidx == ix, neg_inf, cur)
    vk = jnp.concatenate(sel_v, axis=0)
    idx = jnp.concatenate(sel_i, axis=0)

    vals = jnp.bitwise_xor(
        vk, jnp.right_shift(vk, 31) & jnp.int32(0x7FFFFFFF)
    ).view(jnp.float32)

    e = jnp.exp(vals - vals[0:1])
    w = e / jnp.sum(e, axis=0, keepdims=True)

    wi_ref[...] = jnp.concatenate([w, idx.astype(jnp.float32)], axis=0)


@jax.jit
def kernel(hidden_states, weight):
    wt = weight.T
    grid = (_TOKENS // _BT,)
    logits, wi_t = pl.pallas_call(
        _router_body,
        grid=grid,
        in_specs=[
            pl.BlockSpec((_BT, _HIDDEN), lambda i: (i, 0)),
            pl.BlockSpec((_HIDDEN, _E), lambda i: (0, 0)),
        ],
        out_specs=[
            pl.BlockSpec((_BT, _E), lambda i: (i, 0)),
            pl.BlockSpec((2 * _K, _BT), lambda i: (0, i)),
        ],
        out_shape=[
            jax.ShapeDtypeStruct((_TOKENS, _E), jnp.float32),
            jax.ShapeDtypeStruct((2 * _K, _TOKENS), jnp.float32),
        ],
        compiler_params=pltpu.CompilerParams(
            dimension_semantics=("arbitrary",),
        ),
    )(hidden_states, wt)
    wi = wi_t.T
    return (logits, wi[:, :_K], wi[:, _K:].astype(jnp.int32))
